# baseline (speedup 1.0000x reference)
import functools

import jax
import jax.numpy as jnp
from jax import lax
from jax.experimental import pallas as pl
from jax.experimental.pallas import tpu as pltpu
from jax.experimental.pallas import tpu_sc as plsc

N = 10000
NP = 10240
E = 320000
EP = 327680
D_IN = 128
D_HID = 256
D_LAT = 128

NC = 2
NS = 16
LANES = 16
CHUNK = 128
NW = NC * NS
WCH = EP // NW // CHUNK
ROWS_PER_SUB = NP // NS
WB = ROWS_PER_SUB // CHUNK


def _zero_rows(buf, nrows, width):
    zeros = jnp.zeros((LANES,), jnp.float32)

    @pl.loop(0, nrows)
    def _(r):
        for k in range(width // LANES):
            buf[r, pl.ds(k * LANES, LANES)] = zeros


def _deg_body(idx_hbm, deg_hbm, idxv, ones_v, wbuf, acc):
    cid = lax.axis_index("c")
    sid = lax.axis_index("s")
    wid = sid * NC + cid

    zeros = jnp.zeros((LANES,), jnp.float32)
    for k in range(CHUNK // LANES):
        ones_v[pl.ds(k * LANES, LANES)] = jnp.ones((LANES,), jnp.float32)
    for k in range(ROWS_PER_SUB // LANES):
        wbuf[pl.ds(k * LANES, LANES)] = zeros
    pltpu.sync_copy(wbuf, acc.at[pl.ds(sid * ROWS_PER_SUB, ROWS_PER_SUB)])
    plsc.subcore_barrier()

    pltpu.sync_copy(idx_hbm.at[wid], idxv)

    @pl.loop(0, WCH)
    def _(j):
        pltpu.sync_copy(ones_v, acc.at[idxv.at[j, 1]], add=True)

    plsc.subcore_barrier()
    base = sid * ROWS_PER_SUB
    pltpu.sync_copy(acc.at[pl.ds(base, ROWS_PER_SUB)], wbuf)
    pltpu.sync_copy(wbuf, deg_hbm.at[cid, pl.ds(base, ROWS_PER_SUB)])


@functools.cache
def _deg_kernel():
    return functools.partial(
        pl.kernel,
        out_type=jax.ShapeDtypeStruct((NC, NP), jnp.float32),
        mesh=plsc.VectorSubcoreMesh(
            core_axis_name="c", subcore_axis_name="s",
            num_cores=NC, num_subcores=NS),
        scratch_types=[
            pltpu.VMEM((WCH, 2, CHUNK), jnp.int32),
            pltpu.VMEM((CHUNK,), jnp.float32),
            pltpu.VMEM((ROWS_PER_SUB,), jnp.float32),
            pltpu.VMEM_SHARED((NP,), jnp.float32),
        ],
    )(_deg_body)


def _scatter_body(g_hbm, idx_hbm, out_hbm,
                  idxv, buf0, buf1, acc,
                  semi0, semi1, semg0, semg1, Dh=None, edge_split=False):
    cid = lax.axis_index("c")
    sid = lax.axis_index("s")
    if edge_split:
        slices = [sid * NC + cid]
    else:
        slices = [2 * sid, 2 * sid + 1]
    off = cid * NP
    bufs = (buf0, buf1)
    semi = (semi0, semi1)
    semg = (semg0, semg1)

    _zero_rows(buf0, CHUNK, Dh)
    for t in range(WB):
        pltpu.sync_copy(
            buf0, acc.at[pl.ds(sid * ROWS_PER_SUB + t * CHUNK, CHUNK)])
    plsc.subcore_barrier()

    for w in slices:
        def start_idx(j, p):
            pltpu.async_copy(idx_hbm.at[w, j], idxv.at[p], semi[p])

        def finish_idx(j, p):
            pltpu.make_async_copy(idx_hbm.at[w, j], idxv.at[p], semi[p]).wait()
            if not edge_split:
                for k in range(CHUNK // LANES):
                    sl = pl.ds(k * LANES, LANES)
                    idxv[p, 0, sl] = idxv[p, 0, sl] + off

        def start_gather(p):
            pltpu.async_copy(g_hbm.at[idxv.at[p, 0]], bufs[p], semg[p])

        def finish_gather(p):
            pltpu.make_async_copy(
                g_hbm.at[idxv.at[p, 0]], bufs[p], semg[p]).wait()

        def scatter(p):
            pltpu.sync_copy(bufs[p], acc.at[idxv.at[p, 1]], add=True)

        start_idx(0, 0)
        start_idx(1, 1)
        finish_idx(0, 0)
        start_gather(0)

        @pl.loop(0, WCH, step=2)
        def _(j):
            finish_idx(j + 1, 1)
            finish_gather(0)
            start_gather(1)
            scatter(0)

            @pl.when(j + 2 < WCH)
            def _():
                start_idx(j + 2, 0)

            finish_gather(1)

            @pl.when(j + 2 < WCH)
            def _():
                finish_idx(j + 2, 0)
                start_gather(0)

            scatter(1)

            @pl.when(j + 3 < WCH)
            def _():
                start_idx(j + 3, 1)

    plsc.subcore_barrier()

    for t in range(WB):
        base = sid * ROWS_PER_SUB + t * CHUNK
        pltpu.sync_copy(acc.at[pl.ds(base, CHUNK)], buf0)
        pltpu.sync_copy(buf0, out_hbm.at[pl.ds(cid * NP + base, CHUNK)])


@functools.cache
def _make_scatter(Dh, edge_split):
    return functools.partial(
        pl.kernel,
        out_type=jax.ShapeDtypeStruct((2 * NP, Dh), jnp.float32),
        mesh=plsc.VectorSubcoreMesh(
            core_axis_name="c", subcore_axis_name="s",
            num_cores=NC, num_subcores=NS),
        scratch_types=[
            pltpu.VMEM((2, 2, CHUNK), jnp.int32),
            pltpu.VMEM((CHUNK, Dh), jnp.float32),
            pltpu.VMEM((CHUNK, Dh), jnp.float32),
            pltpu.VMEM_SHARED((NP, Dh), jnp.float32),
            pltpu.SemaphoreType.DMA,
            pltpu.SemaphoreType.DMA,
            pltpu.SemaphoreType.DMA,
            pltpu.SemaphoreType.DMA,
        ],
    )(functools.partial(_scatter_body, Dh=Dh, edge_split=edge_split))


BLK = 256
NB = NP // BLK


def _tc_a_body(x_ref, degp_ref, xs_ref, dinv_ref):
    i = pl.program_id(0)
    dinv = lax.rsqrt(degp_ref[0, 0, :] + degp_ref[0, 1, :] + 1.0)
    rows = i * BLK + lax.broadcasted_iota(jnp.int32, (BLK, 1), 0)
    xs_ref[...] = jnp.where(rows < N, x_ref[...] * dinv[:, None], 0.0)
    dinv_ref[...] = dinv[None, None, :]


def _tc_b_body(s_ref, xs_ref, dinv_ref, w1_ref, b1_ref, w2_ref, g2_ref):
    i = pl.program_id(0)
    dinv = dinv_ref[0, 0, :]
    a = s_ref[0] + s_ref[1] + xs_ref[...]
    h = jnp.dot(a, w1_ref[...], preferred_element_type=jnp.float32)
    x2 = jnp.maximum(h * dinv[:, None] + b1_ref[...], 0.0)
    h2 = jnp.dot(x2, w2_ref[...], preferred_element_type=jnp.float32)
    rows = i * BLK + lax.broadcasted_iota(jnp.int32, (BLK, 1), 0)
    g2_ref[...] = h2 * dinv[:, None] * (rows < N).astype(jnp.float32)


CBLK = 80
NCB = N // CBLK


def _tc_c_body(s2_ref, g2_ref, dinv_ref, b2_ref, out_ref):
    dinv = dinv_ref[:, 0]
    s = s2_ref[0] + s2_ref[1] + g2_ref[...]
    out_ref[...] = s * dinv[:, None] + b2_ref[...]


def _row_spec(d):
    return pl.BlockSpec((BLK, d), lambda i: (i, 0))


def _half_spec(d):
    return pl.BlockSpec((2, BLK, d), lambda i: (0, i, 0))


_degp_spec = pl.BlockSpec((1, 2, BLK), lambda i: (i, 0, 0))
_dinv_spec = pl.BlockSpec((1, 1, BLK), lambda i: (i, 0, 0))


def _whole(x):
    return pl.BlockSpec(x.shape, lambda i: tuple(0 for _ in x.shape))


def kernel(x, edge_index, W1, b1, W2, b2):
    f32 = jnp.float32
    row = edge_index[0]
    col = edge_index[1]
    pad = N + jnp.arange(EP - E, dtype=jnp.int32) % (NP - N)
    row_p = jnp.concatenate([row, pad])
    col_p = jnp.concatenate([col, pad])
    idx = jnp.stack(
        [row_p.reshape(NW, WCH, CHUNK), col_p.reshape(NW, WCH, CHUNK)],
        axis=2)
    degp = _deg_kernel()(idx)
    degp3 = degp.reshape(2, NB, BLK).transpose(1, 0, 2)

    xs, dinv2 = pl.pallas_call(
        _tc_a_body,
        grid=(NB,),
        in_specs=[_row_spec(D_IN), _degp_spec],
        out_specs=[_row_spec(D_IN), _dinv_spec],
        out_shape=[jax.ShapeDtypeStruct((NP, D_IN), f32),
                   jax.ShapeDtypeStruct((NB, 1, BLK), f32)],
    )(x, degp3)

    s1 = _make_scatter(D_IN, True)(xs, idx)

    g2 = pl.pallas_call(
        _tc_b_body,
        grid=(NB,),
        in_specs=[_half_spec(D_IN), _row_spec(D_IN), _dinv_spec,
                  _whole(W1), _whole(b1), _whole(W2)],
        out_specs=_row_spec(D_LAT),
        out_shape=jax.ShapeDtypeStruct((NP, D_LAT), f32),
    )(s1.reshape(2, NP, D_IN), xs, dinv2, W1, b1, W2)

    s2 = _make_scatter(D_LAT, True)(g2, idx)

    out = pl.pallas_call(
        _tc_c_body,
        grid=(NCB,),
        in_specs=[pl.BlockSpec((2, CBLK, D_LAT), lambda i: (0, i, 0)),
                  pl.BlockSpec((CBLK, D_LAT), lambda i: (i, 0)),
                  pl.BlockSpec((CBLK, 1), lambda i: (i, 0)),
                  _whole(b2)],
        out_specs=pl.BlockSpec((CBLK, D_LAT), lambda i: (i, 0)),
        out_shape=jax.ShapeDtypeStruct((N, D_LAT), f32),
    )(s2.reshape(2, NP, D_LAT), g2, dinv2.reshape(NP, 1), b2)

    return out


# speedup vs baseline: 1.1020x; 1.1020x over previous
import functools

import jax
import jax.numpy as jnp
from jax import lax
from jax.experimental import pallas as pl
from jax.experimental.pallas import tpu as pltpu
from jax.experimental.pallas import tpu_sc as plsc

N = 10000
NP = 10240
E = 320000
EP = 327680
D_IN = 128
D_HID = 256
D_LAT = 128

NC = 2
NS = 16
LANES = 16
CHUNK = 128
NW = NC * NS
WCH = EP // NW // CHUNK
ROWS_PER_SUB = NP // NS
WB = ROWS_PER_SUB // CHUNK


def _zero_rows(buf, nrows, width):
    zeros = jnp.zeros((LANES,), jnp.float32)

    @pl.loop(0, nrows)
    def _(r):
        for k in range(width // LANES):
            buf[r, pl.ds(k * LANES, LANES)] = zeros


def _deg_body(idx_hbm, deg_hbm, idxv, ones_v, wbuf, acc):
    cid = lax.axis_index("c")
    sid = lax.axis_index("s")
    wid = sid * NC + cid

    zeros = jnp.zeros((LANES,), jnp.float32)
    for k in range(CHUNK // LANES):
        ones_v[pl.ds(k * LANES, LANES)] = jnp.ones((LANES,), jnp.float32)
    for k in range(ROWS_PER_SUB // LANES):
        wbuf[pl.ds(k * LANES, LANES)] = zeros
    pltpu.sync_copy(wbuf, acc.at[pl.ds(sid * ROWS_PER_SUB, ROWS_PER_SUB)])
    plsc.subcore_barrier()

    pltpu.sync_copy(idx_hbm.at[wid], idxv)

    @pl.loop(0, WCH)
    def _(j):
        pltpu.sync_copy(ones_v, acc.at[idxv.at[j, 1]], add=True)

    plsc.subcore_barrier()
    base = sid * ROWS_PER_SUB
    pltpu.sync_copy(acc.at[pl.ds(base, ROWS_PER_SUB)], wbuf)
    pltpu.sync_copy(wbuf, deg_hbm.at[cid, pl.ds(base, ROWS_PER_SUB)])


@functools.cache
def _deg_kernel():
    return functools.partial(
        pl.kernel,
        out_type=jax.ShapeDtypeStruct((NC, NP), jnp.float32),
        mesh=plsc.VectorSubcoreMesh(
            core_axis_name="c", subcore_axis_name="s",
            num_cores=NC, num_subcores=NS),
        scratch_types=[
            pltpu.VMEM((WCH, 2, CHUNK), jnp.int32),
            pltpu.VMEM((CHUNK,), jnp.float32),
            pltpu.VMEM((ROWS_PER_SUB,), jnp.float32),
            pltpu.VMEM_SHARED((NP,), jnp.float32),
        ],
    )(_deg_body)


def _scatter_body(g_hbm, idx_hbm, out_hbm,
                  idxv, buf0, buf1, acc,
                  semi0, semi1, semg0, semg1, Dh=None, edge_split=False):
    cid = lax.axis_index("c")
    sid = lax.axis_index("s")
    if edge_split:
        slices = [sid * NC + cid]
    else:
        slices = [2 * sid, 2 * sid + 1]
    off = cid * NP
    bufs = (buf0, buf1)
    semi = (semi0, semi1)
    semg = (semg0, semg1)

    _zero_rows(buf0, CHUNK, Dh)
    for t in range(WB):
        pltpu.sync_copy(
            buf0, acc.at[pl.ds(sid * ROWS_PER_SUB + t * CHUNK, CHUNK)])
    plsc.subcore_barrier()

    for w in slices:
        def start_idx(j, p):
            pltpu.async_copy(idx_hbm.at[w, j], idxv.at[p], semi[p])

        def finish_idx(j, p):
            pltpu.make_async_copy(idx_hbm.at[w, j], idxv.at[p], semi[p]).wait()
            if not edge_split:
                for k in range(CHUNK // LANES):
                    sl = pl.ds(k * LANES, LANES)
                    idxv[p, 0, sl] = idxv[p, 0, sl] + off

        def start_gather(p):
            pltpu.async_copy(g_hbm.at[idxv.at[p, 0]], bufs[p], semg[p])

        def finish_gather(p):
            pltpu.make_async_copy(
                g_hbm.at[idxv.at[p, 0]], bufs[p], semg[p]).wait()

        def scatter(p):
            pltpu.sync_copy(bufs[p], acc.at[idxv.at[p, 1]], add=True)

        start_idx(0, 0)
        start_idx(1, 1)
        finish_idx(0, 0)
        start_gather(0)

        @pl.loop(0, WCH, step=2)
        def _(j):
            finish_idx(j + 1, 1)
            finish_gather(0)
            start_gather(1)
            scatter(0)

            @pl.when(j + 2 < WCH)
            def _():
                start_idx(j + 2, 0)

            finish_gather(1)

            @pl.when(j + 2 < WCH)
            def _():
                finish_idx(j + 2, 0)
                start_gather(0)

            scatter(1)

            @pl.when(j + 3 < WCH)
            def _():
                start_idx(j + 3, 1)

    plsc.subcore_barrier()

    for t in range(WB):
        base = sid * ROWS_PER_SUB + t * CHUNK
        pltpu.sync_copy(acc.at[pl.ds(base, CHUNK)], buf0)
        pltpu.sync_copy(buf0, out_hbm.at[pl.ds(cid * NP + base, CHUNK)])


@functools.cache
def _make_scatter(Dh, edge_split):
    return functools.partial(
        pl.kernel,
        out_type=jax.ShapeDtypeStruct((2 * NP, Dh), jnp.float32),
        mesh=plsc.VectorSubcoreMesh(
            core_axis_name="c", subcore_axis_name="s",
            num_cores=NC, num_subcores=NS),
        scratch_types=[
            pltpu.VMEM((2, 2, CHUNK), jnp.int32),
            pltpu.VMEM((CHUNK, Dh), jnp.float32),
            pltpu.VMEM((CHUNK, Dh), jnp.float32),
            pltpu.VMEM_SHARED((NP, Dh), jnp.float32),
            pltpu.SemaphoreType.DMA,
            pltpu.SemaphoreType.DMA,
            pltpu.SemaphoreType.DMA,
            pltpu.SemaphoreType.DMA,
        ],
    )(functools.partial(_scatter_body, Dh=Dh, edge_split=edge_split))


BLK = 256
NB = NP // BLK


def _tc_a_body(x_ref, degp_ref, xs_ref, dinv_ref):
    dinv = lax.rsqrt(degp_ref[0, 0, :] + degp_ref[0, 1, :] + 1.0)
    xs_ref[...] = x_ref[...] * dinv[:, None]
    dinv_ref[...] = dinv[None, None, :]


def _tc_b_body(s_ref, xs_ref, dinv_ref, w1_ref, b1_ref, w2_ref, g2_ref):
    i = pl.program_id(0)
    dinv = dinv_ref[0, 0, :]
    a = s_ref[0] + s_ref[1] + xs_ref[...]
    h = jnp.dot(a, w1_ref[...], preferred_element_type=jnp.float32)
    x2 = jnp.maximum(h * dinv[:, None] + b1_ref[...], 0.0)
    h2 = jnp.dot(x2, w2_ref[...], preferred_element_type=jnp.float32)
    rows = i * BLK + lax.broadcasted_iota(jnp.int32, (BLK, 1), 0)
    g2_ref[...] = h2 * dinv[:, None] * (rows < N).astype(jnp.float32)


def _tc_c_body(s2_ref, g2_ref, dinv_ref, b2_ref, out_ref):
    dinv = dinv_ref[0, 0, :]
    s = s2_ref[0] + s2_ref[1] + g2_ref[...]
    out_ref[...] = s * dinv[:, None] + b2_ref[...]


def _row_spec(d):
    return pl.BlockSpec((BLK, d), lambda i: (i, 0))


def _half_spec(d):
    return pl.BlockSpec((2, BLK, d), lambda i: (0, i, 0))


_degp_spec = pl.BlockSpec((1, 2, BLK), lambda i: (i, 0, 0))
_dinv_spec = pl.BlockSpec((1, 1, BLK), lambda i: (i, 0, 0))


def _whole(x):
    return pl.BlockSpec(x.shape, lambda i: tuple(0 for _ in x.shape))


def kernel(x, edge_index, W1, b1, W2, b2):
    f32 = jnp.float32
    row = edge_index[0]
    col = edge_index[1]
    pad = N + jnp.arange(EP - E, dtype=jnp.int32) % (NP - N)
    row_p = jnp.concatenate([row, pad])
    col_p = jnp.concatenate([col, pad])
    idx = jnp.stack(
        [row_p.reshape(NW, WCH, CHUNK), col_p.reshape(NW, WCH, CHUNK)],
        axis=2)
    x_pad = jnp.concatenate([x, jnp.zeros((NP - N, D_IN), f32)])

    degp = _deg_kernel()(idx)
    degp3 = degp.reshape(2, NB, BLK).transpose(1, 0, 2)

    xs, dinv2 = pl.pallas_call(
        _tc_a_body,
        grid=(NB,),
        in_specs=[_row_spec(D_IN), _degp_spec],
        out_specs=[_row_spec(D_IN), _dinv_spec],
        out_shape=[jax.ShapeDtypeStruct((NP, D_IN), f32),
                   jax.ShapeDtypeStruct((NB, 1, BLK), f32)],
    )(x_pad, degp3)

    s1 = _make_scatter(D_IN, True)(xs, idx)

    g2 = pl.pallas_call(
        _tc_b_body,
        grid=(NB,),
        in_specs=[_half_spec(D_IN), _row_spec(D_IN), _dinv_spec,
                  _whole(W1), _whole(b1), _whole(W2)],
        out_specs=_row_spec(D_LAT),
        out_shape=jax.ShapeDtypeStruct((NP, D_LAT), f32),
    )(s1.reshape(2, NP, D_IN), xs, dinv2, W1, b1, W2)

    s2 = _make_scatter(D_LAT, True)(g2, idx)

    out = pl.pallas_call(
        _tc_c_body,
        grid=(NB,),
        in_specs=[_half_spec(D_LAT), _row_spec(D_LAT), _dinv_spec,
                  _whole(b2)],
        out_specs=_row_spec(D_LAT),
        out_shape=jax.ShapeDtypeStruct((NP, D_LAT), f32),
    )(s2.reshape(2, NP, D_LAT), g2, dinv2, b2)

    return out[:N]
